# baseline (speedup 1.0000x reference)
import jax
import jax.numpy as jnp
from jax import lax
from jax.experimental import pallas as pl
from jax.experimental.pallas import tpu as pltpu
from jax.experimental.pallas import tpu_sc as plsc

SEQ = 200
DIM = 32
NW = 32
BBLK = 128


def _sc_body(idx_hbm, word_hbm, pos_hbm, out_hbm,
             idx_bw, idx_w, pos_v, rows_v, tr_v, gat_sem, out_sem):
    nc = 2
    wid = lax.axis_index("s") * nc + lax.axis_index("c")
    iota = lax.iota(jnp.int32, 16)

    pltpu.sync_copy(idx_hbm.at[pl.ds(wid * BBLK, BBLK)], idx_bw)
    pltpu.sync_copy(pos_hbm, pos_v)

    def tr_idx(s, _):
        sv = jnp.full((16,), s, jnp.int32)
        for j0 in range(BBLK // 16):
            v = plsc.load_gather(idx_bw, [j0 * 16 + iota, sv])
            idx_w[s, pl.ds(j0 * 16, 16)] = v
        return _

    lax.fori_loop(0, SEQ, tr_idx, None)

    def gat_copy(s, b):
        return pltpu.make_async_copy(
            word_hbm.at[idx_w.at[s]], rows_v.at[b], gat_sem.at[b])

    def out_copies(s, b):
        return [pltpu.make_async_copy(
            tr_v.at[b, pl.ds(tr * 8, 8)], out_hbm.at[s, tr, wid],
            out_sem.at[b]) for tr in range(4)]

    gat_copy(0, 0).start()

    def pair_body(i, _):
        for b in (0, 1):
            s = 2 * i + b
            gat_copy(s, b).wait()

            @pl.when(s + 1 < SEQ)
            def _():
                gat_copy(s + 1, 1 - b).start()

            @pl.when(s >= 2)
            def _():
                for c in out_copies(s - 2, b):
                    c.wait()

            rb = rows_v.at[b]
            tb = tr_v.at[b]
            sv = jnp.full((16,), s, jnp.int32)
            for c in range(DIM):
                cv = jnp.full((16,), c, jnp.int32)
                pvec = plsc.load_gather(pos_v, [sv, cv])
                for j0 in range(BBLK // 16):
                    v = plsc.load_gather(rb, [j0 * 16 + iota, cv])
                    tb[c, pl.ds(j0 * 16, 16)] = v + pvec
            for cpy in out_copies(s, b):
                cpy.start()
        return _

    lax.fori_loop(0, SEQ // 2, pair_body, None)
    for cpy in out_copies(SEQ - 2, 0) + out_copies(SEQ - 1, 1):
        cpy.wait()


def kernel(inputs, word_table, pos_table):
    bsz, s = inputs.shape
    idx2d = inputs.astype(jnp.int32)
    mesh = plsc.VectorSubcoreMesh(core_axis_name="c", subcore_axis_name="s")
    out5 = pl.kernel(
        _sc_body,
        out_type=jax.ShapeDtypeStruct((SEQ, 4, NW, 8, BBLK), jnp.float32),
        mesh=mesh,
        compiler_params=pltpu.CompilerParams(
            use_tc_tiling_on_sc=False, needs_layout_passes=False),
        scratch_types=[
            pltpu.VMEM((BBLK, SEQ), jnp.int32),
            pltpu.VMEM((SEQ, BBLK), jnp.int32),
            pltpu.VMEM((SEQ, DIM), jnp.float32),
            pltpu.VMEM((2, BBLK, DIM), jnp.float32),
            pltpu.VMEM((2, DIM, BBLK), jnp.float32),
            pltpu.SemaphoreType.DMA((2,)),
            pltpu.SemaphoreType.DMA((2,)),
        ],
    )(idx2d, word_table, pos_table)
    return out5.transpose(2, 4, 0, 1, 3).reshape(bsz, s, DIM)
